# baseline (speedup 1.0000x reference)
import functools

import jax
import jax.numpy as jnp
from jax import lax
from jax.experimental import pallas as pl
from jax.experimental.pallas import tpu as pltpu
from jax.experimental.pallas import tpu_sc as plsc

_DIMS = [119, 5, 12, 12, 10, 6, 6, 2, 2]
_NT = len(_DIMS)
_OFFS = [sum(_DIMS[:i]) for i in range(_NT)]
_ROWS = sum(_DIMS)
_D = 128
_N = 100000
_BR = 32
_NBLK = _N // _BR
_NW = 32
_BASE = _NBLK // _NW
_REM = _NBLK % _NW
_CH = (_BASE + 1) * _BR * _NT

_mesh = plsc.VectorSubcoreMesh(
    core_axis_name="c", subcore_axis_name="s", num_cores=2, num_subcores=16
)

_GROUPS = [
    (5, 12, _OFFS[1], _OFFS[2], 119),
    (12, 10, _OFFS[3], _OFFS[4], 179),
]
_QBASE = 299
_CROWS = 443


@functools.partial(
    pl.kernel,
    out_type=jax.ShapeDtypeStruct((_N, _D), jnp.float32),
    mesh=_mesh,
    scratch_types=[
        pltpu.VMEM((_ROWS * _D,), jnp.float32),
        pltpu.VMEM((_CROWS * _D,), jnp.float32),
        pltpu.VMEM((_CH + 16,), jnp.int32),
        pltpu.VMEM((_BR, _D), jnp.float32),
        pltpu.VMEM((_BR, _D), jnp.float32),
        pltpu.VMEM((4 * _D,), jnp.float32),
        pltpu.VMEM((_BR,), jnp.int32),
        pltpu.VMEM((_BR,), jnp.int32),
        pltpu.SemaphoreType.DMA,
        pltpu.SemaphoreType.DMA,
        pltpu.SemaphoreType.DMA,
        pltpu.SemaphoreType.DMA,
        pltpu.SemaphoreType.DMA,
    ],
    compiler_params=pltpu.CompilerParams(
        use_tc_tiling_on_sc=False, needs_layout_passes=False
    ),
)
def _sc_embed(
    xf,
    wf,
    w0m,
    out,
    raw_v,
    comb_v,
    xch,
    ovA,
    ovB,
    t4_v,
    idxA,
    idxB,
    sem_x,
    sem_ga,
    sem_oa,
    sem_gb,
    sem_ob,
):
    wid = lax.axis_index("s") * 2 + lax.axis_index("c")
    nblocks = jnp.where(wid < _REM, _BASE + 1, _BASE)
    b0 = wid * _BASE + jnp.minimum(wid, _REM)

    @pl.when(nblocks == _BASE + 1)
    def _():
        pltpu.async_copy(
            xf.at[pl.ds(b0 * _BR * _NT, (_BASE + 1) * _BR * _NT)],
            xch.at[pl.ds(0, (_BASE + 1) * _BR * _NT)],
            sem_x,
        )

    @pl.when(nblocks == _BASE)
    def _():
        pltpu.async_copy(
            xf.at[pl.ds(b0 * _BR * _NT, _BASE * _BR * _NT)],
            xch.at[pl.ds(0, _BASE * _BR * _NT)],
            sem_x,
        )

    pltpu.sync_copy(wf, raw_v)

    for dA, dB, offA, offB, gbase in _GROUPS:

        @pl.loop(0, dA)
        def _a(a, _dB=dB, _offA=offA, _offB=offB, _gbase=gbase):
            va = [raw_v[pl.ds((_offA + a) * _D + k * 16, 16)] for k in range(8)]
            rowbase = (_gbase + a * _dB) * _D

            @plsc.parallel_loop(0, _dB, unroll=2)
            def _b(b):
                src = (_offB + b) * _D
                dst = rowbase + b * _D
                for k in range(8):
                    comb_v[pl.ds(dst + k * 16, 16)] = va[k] + raw_v[
                        pl.ds(src + k * 16, 16)
                    ]

    for c in range(2):
        for d in range(2):
            for k in range(8):
                t4_v[pl.ds((c * 2 + d) * _D + k * 16, 16)] = (
                    raw_v[pl.ds((_OFFS[7] + c) * _D + k * 16, 16)]
                    + raw_v[pl.ds((_OFFS[8] + d) * _D + k * 16, 16)]
                )

    @pl.loop(0, 6)
    def _qa(a):
        va = [raw_v[pl.ds((_OFFS[5] + a) * _D + k * 16, 16)] for k in range(8)]

        @plsc.parallel_loop(0, 6, unroll=2)
        def _qb(b):
            vab = [
                va[k] + raw_v[pl.ds((_OFFS[6] + b) * _D + k * 16, 16)]
                for k in range(8)
            ]
            base = (_QBASE + (a * 6 + b) * 4) * _D
            for cd in range(4):
                for k in range(8):
                    comb_v[pl.ds(base + cd * _D + k * 16, 16)] = vab[k] + t4_v[
                        pl.ds(cd * _D + k * 16, 16)
                    ]

    @pl.when(nblocks == _BASE + 1)
    def _():
        pltpu.make_async_copy(
            xf.at[pl.ds(b0 * _BR * _NT, (_BASE + 1) * _BR * _NT)],
            xch.at[pl.ds(0, (_BASE + 1) * _BR * _NT)],
            sem_x,
        ).wait()

    @pl.when(nblocks == _BASE)
    def _():
        pltpu.make_async_copy(
            xf.at[pl.ds(b0 * _BR * _NT, _BASE * _BR * _NT)],
            xch.at[pl.ds(0, _BASE * _BR * _NT)],
            sem_x,
        ).wait()

    iota9 = lax.broadcasted_iota(jnp.int32, (16,), 0) * _NT

    def compute_block(lb, oref, idxref):
        for h in range(2):
            idxref[pl.ds(h * 16, 16)] = plsc.load_gather(
                xch, [iota9 + (lb + h * 16) * _NT]
            )

        @plsc.parallel_loop(0, _BR, unroll=4)
        def _row(r):
            lr = lb + r
            xrow = xch[pl.ds(lr * _NT, 16)]
            bases = [
                (xrow[1] * 12 + xrow[2] + 119) * _D,
                (xrow[3] * 10 + xrow[4] + 179) * _D,
                ((xrow[5] * 6 + xrow[6]) * 4 + xrow[7] * 2 + xrow[8] + _QBASE)
                * _D,
            ]
            for j0 in range(0, _D, 16):
                acc = comb_v[pl.ds(bases[0] + j0, 16)]
                acc = acc + comb_v[pl.ds(bases[1] + j0, 16)]
                acc = acc + comb_v[pl.ds(bases[2] + j0, 16)]
                oref[r, pl.ds(j0, 16)] = acc

    def wait_gadd(oref, idxref, sem):
        pltpu.make_async_copy(w0m.at[idxref], oref, sem).wait()

    def wait_odma(oref, sem):
        pltpu.make_async_copy(oref, out.at[pl.ds(b0 * _BR, _BR)], sem).wait()

    nb_even = (nblocks // 2) * 2

    @pl.loop(0, nb_even, step=2)
    def _pair(t2):
        b = b0 + t2

        @pl.when(t2 > 0)
        def _():
            wait_odma(ovA, sem_oa)

        compute_block(t2 * _BR, ovA, idxA)

        @pl.when(t2 > 0)
        def _():
            wait_gadd(ovB, idxB, sem_gb)
            pltpu.async_copy(ovB, out.at[pl.ds((b - 1) * _BR, _BR)], sem_ob)

        pltpu.async_copy(w0m.at[idxA], ovA, sem_ga, add=True)

        @pl.when(t2 > 0)
        def _():
            wait_odma(ovB, sem_ob)

        compute_block((t2 + 1) * _BR, ovB, idxB)
        wait_gadd(ovA, idxA, sem_ga)
        pltpu.async_copy(ovA, out.at[pl.ds(b * _BR, _BR)], sem_oa)
        pltpu.async_copy(w0m.at[idxB], ovB, sem_gb, add=True)

    wait_gadd(ovB, idxB, sem_gb)
    pltpu.async_copy(
        ovB, out.at[pl.ds((b0 + nb_even - 1) * _BR, _BR)], sem_ob
    )

    @pl.when(nblocks > nb_even)
    def _():
        tb = nb_even
        wait_odma(ovA, sem_oa)
        compute_block(tb * _BR, ovA, idxA)
        pltpu.async_copy(w0m.at[idxA], ovA, sem_ga, add=True)
        wait_gadd(ovA, idxA, sem_ga)
        pltpu.async_copy(ovA, out.at[pl.ds((b0 + tb) * _BR, _BR)], sem_oa)

    wait_odma(ovA, sem_oa)
    wait_odma(ovB, sem_ob)


def kernel(x, W0, W1, W2, W3, W4, W5, W6, W7, W8):
    wf = jnp.concatenate([W0, W1, W2, W3, W4, W5, W6, W7, W8], axis=0).reshape(-1)
    xf = x.reshape(-1)
    return _sc_embed(xf, wf, W0)


# speedup vs baseline: 1.4488x; 1.4488x over previous
import functools

import jax
import jax.numpy as jnp
from jax import lax
from jax.experimental import pallas as pl
from jax.experimental.pallas import tpu as pltpu
from jax.experimental.pallas import tpu_sc as plsc

_DIMS = [119, 5, 12, 12, 10, 6, 6, 2, 2]
_NT = len(_DIMS)
_OFFS = [sum(_DIMS[:i]) for i in range(_NT)]
_ROWS = sum(_DIMS)
_D = 128
_N = 100000
_BR = 32
_NBLK = _N // _BR
_NW = 32
_BASE = _NBLK // _NW
_REM = _NBLK % _NW
_MAXR = (_BASE + 1) * _BR
_NPAD = _NW * _BASE * _BR + _REM * _BR + _BR
_CH = _MAXR * _NT

_mesh = plsc.VectorSubcoreMesh(
    core_axis_name="c", subcore_axis_name="s", num_cores=2, num_subcores=16
)

_GROUPS = [
    (5, 12, _OFFS[1], _OFFS[2], 119),
    (12, 10, _OFFS[3], _OFFS[4], 179),
]
_QBASE = 299
_CROWS = 443


@functools.partial(
    pl.kernel,
    out_type=jax.ShapeDtypeStruct((_N, _D), jnp.float32),
    mesh=_mesh,
    scratch_types=[
        pltpu.VMEM((_ROWS * _D,), jnp.float32),
        pltpu.VMEM((_CROWS * _D,), jnp.float32),
        pltpu.VMEM((_CH + 16,), jnp.int32),
        pltpu.VMEM((16, _D), jnp.float32),
        pltpu.VMEM((16, _D), jnp.float32),
        pltpu.VMEM((4 * _D,), jnp.float32),
        pltpu.SemaphoreType.DMA,
        pltpu.SemaphoreType.DMA,
        pltpu.SemaphoreType.DMA,
        pltpu.SemaphoreType.DMA,
    ],
    compiler_params=pltpu.CompilerParams(
        use_tc_tiling_on_sc=False, needs_layout_passes=False
    ),
)
def _sc_embed(
    xf, wf, out, raw_v, comb_v, xch, ov0, ov1, t4_v, sem_x, sem_g0, sem_o0, sem_o1
):
    wid = lax.axis_index("s") * 2 + lax.axis_index("c")
    nblocks = jnp.where(wid < _REM, _BASE + 1, _BASE)
    b0 = wid * _BASE + jnp.minimum(wid, _REM)

    @pl.when(nblocks == _BASE + 1)
    def _():
        pltpu.async_copy(
            xf.at[pl.ds(b0 * _BR * _NT, (_BASE + 1) * _BR * _NT)],
            xch.at[pl.ds(0, (_BASE + 1) * _BR * _NT)],
            sem_x,
        )

    @pl.when(nblocks == _BASE)
    def _():
        pltpu.async_copy(
            xf.at[pl.ds(b0 * _BR * _NT, _BASE * _BR * _NT)],
            xch.at[pl.ds(0, _BASE * _BR * _NT)],
            sem_x,
        )
    pltpu.async_copy(
        wf.at[pl.ds(0, 119 * _D)], comb_v.at[pl.ds(0, 119 * _D)], sem_g0
    )
    pltpu.sync_copy(wf, raw_v)

    for dA, dB, offA, offB, gbase in _GROUPS:

        @pl.loop(0, dA)
        def _a(a, _dB=dB, _offA=offA, _offB=offB, _gbase=gbase):
            va = [raw_v[pl.ds((_offA + a) * _D + k * 16, 16)] for k in range(8)]
            rowbase = (_gbase + a * _dB) * _D

            @plsc.parallel_loop(0, _dB, unroll=2)
            def _b(b):
                src = (_offB + b) * _D
                dst = rowbase + b * _D
                for k in range(8):
                    comb_v[pl.ds(dst + k * 16, 16)] = va[k] + raw_v[
                        pl.ds(src + k * 16, 16)
                    ]

    for c in range(2):
        for d in range(2):
            for k in range(8):
                t4_v[pl.ds((c * 2 + d) * _D + k * 16, 16)] = (
                    raw_v[pl.ds((_OFFS[7] + c) * _D + k * 16, 16)]
                    + raw_v[pl.ds((_OFFS[8] + d) * _D + k * 16, 16)]
                )

    @pl.loop(0, 6)
    def _qa(a):
        va = [raw_v[pl.ds((_OFFS[5] + a) * _D + k * 16, 16)] for k in range(8)]

        @plsc.parallel_loop(0, 6, unroll=2)
        def _qb(b):
            vab = [
                va[k] + raw_v[pl.ds((_OFFS[6] + b) * _D + k * 16, 16)]
                for k in range(8)
            ]
            base = (_QBASE + (a * 6 + b) * 4) * _D
            for cd in range(4):
                for k in range(8):
                    comb_v[pl.ds(base + cd * _D + k * 16, 16)] = vab[k] + t4_v[
                        pl.ds(cd * _D + k * 16, 16)
                    ]

    pltpu.make_async_copy(
        wf.at[pl.ds(0, 119 * _D)], comb_v.at[pl.ds(0, 119 * _D)], sem_g0
    ).wait()

    @pl.when(nblocks == _BASE + 1)
    def _():
        pltpu.make_async_copy(
            xf.at[pl.ds(b0 * _BR * _NT, (_BASE + 1) * _BR * _NT)],
            xch.at[pl.ds(0, (_BASE + 1) * _BR * _NT)],
            sem_x,
        ).wait()

    @pl.when(nblocks == _BASE)
    def _():
        pltpu.make_async_copy(
            xf.at[pl.ds(b0 * _BR * _NT, _BASE * _BR * _NT)],
            xch.at[pl.ds(0, _BASE * _BR * _NT)],
            sem_x,
        ).wait()

    def compute_half(local_r, oref):
        @plsc.parallel_loop(0, 16, unroll=4)
        def _row(r):
            lr = local_r + r
            xrow = xch[pl.ds(lr * _NT, 16)]
            bases = [
                xrow[0] * _D,
                (xrow[1] * 12 + xrow[2] + 119) * _D,
                (xrow[3] * 10 + xrow[4] + 179) * _D,
                ((xrow[5] * 6 + xrow[6]) * 4 + xrow[7] * 2 + xrow[8] + _QBASE) * _D,
            ]
            for j0 in range(0, _D, 16):
                acc = comb_v[pl.ds(bases[0] + j0, 16)]
                for g in range(1, 4):
                    acc = acc + comb_v[pl.ds(bases[g] + j0, 16)]
                oref[r, pl.ds(j0, 16)] = acc

    @pl.loop(0, nblocks)
    def _pair(t):
        b = b0 + t
        for half, (oref, sem) in enumerate(((ov0, sem_o0), (ov1, sem_o1))):
            @pl.when(t >= 1)
            def _():
                pltpu.make_async_copy(
                    oref, out.at[pl.ds((b - 1) * _BR + half * 16, 16)], sem
                ).wait()

            compute_half(t * _BR + half * 16, oref)
            pltpu.async_copy(oref, out.at[pl.ds(b * _BR + half * 16, 16)], sem)

    blast = b0 + nblocks - 1
    for half, (oref, sem) in enumerate(((ov0, sem_o0), (ov1, sem_o1))):
        pltpu.make_async_copy(
            oref, out.at[pl.ds(blast * _BR + half * 16, 16)], sem
        ).wait()


def kernel(x, W0, W1, W2, W3, W4, W5, W6, W7, W8):
    wf = jnp.concatenate([W0, W1, W2, W3, W4, W5, W6, W7, W8], axis=0).reshape(-1)
    xf = x.reshape(-1)
    return _sc_embed(xf, wf)
